# baseline (speedup 1.0000x reference)
import functools

import jax
import jax.numpy as jnp
from jax import lax
from jax.experimental import pallas as pl
from jax.experimental.pallas import tpu as pltpu
from jax.experimental.pallas import tpu_sc as plsc

_NE = 8
_D = 1024
_F = 2816
_S = 2048
_K = 2
_BLK = 256
_NB = (_K * _S) // _BLK + (_NE - 1)
_P = _NB * _BLK
_NEG = -1e30


def _route_body(x_ref, rw_ref, dest_ref, wts_ref, be_ref, bv_ref):
    xf = x_ref[...]
    rw = rw_ref[...]
    logits = jax.lax.dot_general(
        rw, xf, (((1,), (1,)), ((), ())),
        preferred_element_type=jnp.float32)
    iota_e = jax.lax.broadcasted_iota(jnp.int32, (_NE, _S), 0)
    v1 = jnp.max(logits, axis=0, keepdims=True)
    i1 = jnp.min(jnp.where(logits == v1, iota_e, _NE), axis=0, keepdims=True)
    m1 = (iota_e == i1)
    l2 = jnp.where(m1, _NEG, logits)
    v2 = jnp.max(l2, axis=0, keepdims=True)
    i2 = jnp.min(jnp.where(l2 == v2, iota_e, _NE), axis=0, keepdims=True)
    m2 = (iota_e == i2)
    a = jnp.exp(v2 - v1)
    w0 = 1.0 / (1.0 + a)

    m1f = m1.astype(jnp.float32)
    m2f = m2.astype(jnp.float32)
    ti = jax.lax.broadcasted_iota(jnp.int32, (_S, _S), 0)
    tj = jax.lax.broadcasted_iota(jnp.int32, (_S, _S), 1)
    lt = (ti < tj).astype(jnp.float32)
    c2 = jnp.concatenate([m1f, m2f], axis=0)
    p12 = jax.lax.dot_general(
        c2, lt, (((1,), (0,)), ((), ())),
        preferred_element_type=jnp.float32)
    p1 = p12[:_NE]
    p2 = p12[_NE:]
    counts1 = jnp.sum(m1f, axis=1, keepdims=True)
    counts2 = jnp.sum(m2f, axis=1, keepdims=True)
    c = counts1 + counts2
    nb = jnp.ceil(c / _BLK)
    ei = jax.lax.broadcasted_iota(jnp.int32, (_NE, _NE), 0)
    ej = jax.lax.broadcasted_iota(jnp.int32, (_NE, _NE), 1)
    lt8 = (ej < ei).astype(jnp.float32)
    snb = jax.lax.dot_general(
        lt8, nb, (((1,), (0,)), ((), ())),
        preferred_element_type=jnp.float32)
    estart = _BLK * snb

    rank0 = jnp.sum(m1f * p1, axis=0, keepdims=True)
    rank1 = jnp.sum(m2f * (p2 + counts1), axis=0, keepdims=True)
    base0 = jnp.sum(m1f * estart, axis=0, keepdims=True)
    base1 = jnp.sum(m2f * estart, axis=0, keepdims=True)
    dest_ref[0:1, :] = (base0 + rank0).astype(jnp.int32)
    dest_ref[1:2, :] = (base1 + rank1).astype(jnp.int32)
    wts_ref[0:1, :] = w0
    wts_ref[1:2, :] = 1.0 - w0

    ends = (snb + nb) * 1.0
    bio = jax.lax.broadcasted_iota(jnp.int32, (_NE, 128), 1).astype(jnp.float32)
    cnt = jnp.sum((bio >= ends).astype(jnp.float32), axis=0, keepdims=True)
    be_ref[...] = jnp.minimum(cnt, _NE - 1.0).astype(jnp.int32)
    total = jnp.sum(nb, axis=0, keepdims=True)
    bv_ref[...] = (bio[0:1, :] < total).astype(jnp.int32)


@jax.jit
def _route(x, router_w):
    return pl.pallas_call(
        _route_body,
        grid=(1,),
        in_specs=[
            pl.BlockSpec((_S, _D), lambda i: (0, 0)),
            pl.BlockSpec((_NE, _D), lambda i: (0, 0)),
        ],
        out_specs=[
            pl.BlockSpec((2, _S), lambda i: (0, 0)),
            pl.BlockSpec((2, _S), lambda i: (0, 0)),
            pl.BlockSpec((1, 128), lambda i: (0, 0)),
            pl.BlockSpec((1, 128), lambda i: (0, 0)),
        ],
        out_shape=[
            jax.ShapeDtypeStruct((2, _S), jnp.int32),
            jax.ShapeDtypeStruct((2, _S), jnp.float32),
            jax.ShapeDtypeStruct((1, 128), jnp.int32),
            jax.ShapeDtypeStruct((1, 128), jnp.int32),
        ],
    )(x, router_w)


_FH = _F // 2


def _expert_half(xs_ref, gw_ref, uw_ref, dw_ref):
    xb = xs_ref[...].astype(jnp.bfloat16)
    g = jax.lax.dot_general(
        xb, gw_ref[0].astype(jnp.bfloat16), (((1,), (1,)), ((), ())),
        preferred_element_type=jnp.float32)
    u = jax.lax.dot_general(
        xb, uw_ref[0].astype(jnp.bfloat16), (((1,), (1,)), ((), ())),
        preferred_element_type=jnp.float32)
    h = (g * jax.nn.sigmoid(g) * u).astype(jnp.bfloat16)
    return jax.lax.dot_general(
        h, dw_ref[0].astype(jnp.bfloat16), (((1,), (1,)), ((), ())),
        preferred_element_type=jnp.float32)


def _ffn1_body(be_ref, bv_ref, xs_ref, gw_ref, uw_ref, dw_ref, part_ref):
    b = pl.program_id(0)

    @pl.when(bv_ref[b] == 1)
    def _():
        part_ref[...] = _expert_half(
            xs_ref, gw_ref, uw_ref, dw_ref).astype(jnp.bfloat16)


def _ffn2_body(be_ref, bv_ref, xs_ref, gw_ref, uw_ref, dw_ref, part_ref,
               yp_ref):
    b = pl.program_id(0)

    @pl.when(bv_ref[b] == 1)
    def _():
        eo = _expert_half(xs_ref, gw_ref, uw_ref, dw_ref)
        yp_ref[...] = part_ref[...].astype(jnp.float32) + eo


def _ffn_specs(f, extra_in=()):
    return pltpu.PrefetchScalarGridSpec(
        num_scalar_prefetch=2,
        grid=(_NB,),
        in_specs=[
            pl.BlockSpec((_BLK, _D), lambda b, be, bv: (b, 0)),
            pl.BlockSpec((1, _FH, _D), lambda b, be, bv, f=f: (be[b], f, 0)),
            pl.BlockSpec((1, _FH, _D), lambda b, be, bv, f=f: (be[b], f, 0)),
            pl.BlockSpec((1, _D, _FH), lambda b, be, bv, f=f: (be[b], 0, f)),
        ] + list(extra_in),
        out_specs=pl.BlockSpec((_BLK, _D), lambda b, be, bv: (b, 0)),
    )


@jax.jit
def _ffn(xs, gw, uw, dw, be, bv):
    part = pl.pallas_call(
        _ffn1_body,
        grid_spec=_ffn_specs(0),
        out_shape=jax.ShapeDtypeStruct((_P, _D), jnp.bfloat16),
        compiler_params=pltpu.CompilerParams(
            dimension_semantics=("arbitrary",),
        ),
    )(be, bv, xs, gw, uw, dw)
    return pl.pallas_call(
        _ffn2_body,
        grid_spec=_ffn_specs(
            1, [pl.BlockSpec((_BLK, _D), lambda b, be, bv: (b, 0))]),
        out_shape=jax.ShapeDtypeStruct((_P, _D), jnp.float32),
        compiler_params=pltpu.CompilerParams(
            dimension_semantics=("arbitrary",),
        ),
    )(be, bv, xs, gw, uw, dw, part)


_MESH = plsc.VectorSubcoreMesh(core_axis_name="c", subcore_axis_name="s")


def _wid():
    return lax.axis_index("s") * 2 + lax.axis_index("c")


def _dispatch_body(x_hbm, dest_hbm, xs_hbm, idx_v, buf0, buf1,
                   rs0, rs1, ws0, ws1):
    w = _wid()
    k = w // 16
    t0 = (w % 16) * 128
    for j in range(4):
        pltpu.sync_copy(dest_hbm.at[k, pl.ds(t0 + j * 32, 32)], idx_v.at[j])
    bufs = (buf0, buf1)
    rsems = (rs0, rs1)
    wsems = (ws0, ws1)
    rd = [None] * 4
    wr = [None] * 4
    rd[0] = pltpu.async_copy(x_hbm.at[pl.ds(t0, 32)], buf0, rs0)
    rd[1] = pltpu.async_copy(x_hbm.at[pl.ds(t0 + 32, 32)], buf1, rs1)
    for j in range(4):
        rd[j].wait()
        wr[j] = pltpu.async_copy(bufs[j % 2], xs_hbm.at[idx_v.at[j]],
                                 wsems[j % 2])
        if j + 2 < 4:
            wr[j].wait()
            rd[j + 2] = pltpu.async_copy(
                x_hbm.at[pl.ds(t0 + (j + 2) * 32, 32)], bufs[j % 2],
                rsems[j % 2])
    wr[2].wait()
    wr[3].wait()


@jax.jit
def _dispatch(x, dest):
    f = functools.partial(
        pl.kernel, mesh=_MESH,
        out_type=jax.ShapeDtypeStruct((_P, _D), jnp.float32),
        scratch_types=[
            pltpu.VMEM((4, 32), jnp.int32),
            pltpu.VMEM((32, _D), jnp.float32),
            pltpu.VMEM((32, _D), jnp.float32),
            pltpu.SemaphoreType.DMA,
            pltpu.SemaphoreType.DMA,
            pltpu.SemaphoreType.DMA,
            pltpu.SemaphoreType.DMA,
        ],
    )(_dispatch_body)
    return f(x, dest)


def _gather2_body(yp_hbm, dest_hbm, r0_hbm, r1_hbm, idx_v, buf0, buf1,
                  rs0, rs1, ws0, ws1):
    w = _wid()
    t0 = w * 64
    for u in range(4):
        k, j = u // 2, u % 2
        pltpu.sync_copy(dest_hbm.at[k, pl.ds(t0 + j * 32, 32)], idx_v.at[u])
    bufs = (buf0, buf1)
    rsems = (rs0, rs1)
    wsems = (ws0, ws1)
    outs = (r0_hbm, r0_hbm, r1_hbm, r1_hbm)
    rd = [None] * 4
    wr = [None] * 4
    rd[0] = pltpu.async_copy(yp_hbm.at[idx_v.at[0]], buf0, rs0)
    rd[1] = pltpu.async_copy(yp_hbm.at[idx_v.at[1]], buf1, rs1)
    for u in range(4):
        j = u % 2
        rd[u].wait()
        wr[u] = pltpu.async_copy(bufs[u % 2],
                                 outs[u].at[pl.ds(t0 + j * 32, 32)],
                                 wsems[u % 2])
        if u + 2 < 4:
            wr[u].wait()
            rd[u + 2] = pltpu.async_copy(yp_hbm.at[idx_v.at[u + 2]],
                                         bufs[u % 2], rsems[u % 2])
    wr[2].wait()
    wr[3].wait()


@jax.jit
def _gather2(yp, dest):
    f = functools.partial(
        pl.kernel, mesh=_MESH,
        out_type=[
            jax.ShapeDtypeStruct((_S, _D), jnp.float32),
            jax.ShapeDtypeStruct((_S, _D), jnp.float32),
        ],
        scratch_types=[
            pltpu.VMEM((4, 32), jnp.int32),
            pltpu.VMEM((32, _D), jnp.float32),
            pltpu.VMEM((32, _D), jnp.float32),
            pltpu.SemaphoreType.DMA,
            pltpu.SemaphoreType.DMA,
            pltpu.SemaphoreType.DMA,
            pltpu.SemaphoreType.DMA,
        ],
    )(_gather2_body)
    return f(yp, dest)


def _mix_body(x_ref, rw_ref, r0_ref, r1_ref, out_ref):
    logits = jax.lax.dot_general(
        x_ref[...], rw_ref[...], (((1,), (1,)), ((), ())),
        preferred_element_type=jnp.float32)
    iota = jax.lax.broadcasted_iota(jnp.int32, logits.shape, 1)
    v1 = jnp.max(logits, axis=1, keepdims=True)
    i1 = jnp.min(jnp.where(logits == v1, iota, _NE), axis=1, keepdims=True)
    l2 = jnp.where(iota == i1, _NEG, logits)
    v2 = jnp.max(l2, axis=1, keepdims=True)
    a = jnp.exp(v2 - v1)
    w0 = 1.0 / (1.0 + a)
    out_ref[...] = w0 * r0_ref[...] + (1.0 - w0) * r1_ref[...]


@jax.jit
def _mix(x, router_w, r0, r1):
    tb = 512
    return pl.pallas_call(
        _mix_body,
        grid=(_S // tb,),
        in_specs=[
            pl.BlockSpec((tb, _D), lambda t: (t, 0)),
            pl.BlockSpec((_NE, _D), lambda t: (0, 0)),
            pl.BlockSpec((tb, _D), lambda t: (t, 0)),
            pl.BlockSpec((tb, _D), lambda t: (t, 0)),
        ],
        out_specs=pl.BlockSpec((tb, _D), lambda t: (t, 0)),
        out_shape=jax.ShapeDtypeStruct((_S, _D), jnp.float32),
    )(x, router_w, r0, r1)


def kernel(hidden_states, router_w, gate_w, up_w, down_w):
    B, S, D = hidden_states.shape
    x = hidden_states.reshape(S, D)
    dest, wts, be, bv = _route(x, router_w)
    be1 = be.reshape(128)
    bv1 = bv.reshape(128)
    xs = _dispatch(x, dest)
    yp = _ffn(xs, gate_w, up_w, down_w, be1, bv1)
    r0, r1 = _gather2(yp, dest)
    out = _mix(x, router_w, r0, r1)
    return out.reshape(B, S, D)


# speedup vs baseline: 1.1147x; 1.1147x over previous
import functools

import jax
import jax.numpy as jnp
from jax import lax
from jax.experimental import pallas as pl
from jax.experimental.pallas import tpu as pltpu
from jax.experimental.pallas import tpu_sc as plsc

_NE = 8
_D = 1024
_F = 2816
_S = 2048
_K = 2
_BLK = 512
_NB = (_K * _S) // _BLK + (_NE - 1)
_P = _NB * _BLK
_NEG = -1e30


def _route_body(x_ref, rw_ref, dest_ref, wts_ref, be_ref, bv_ref):
    xf = x_ref[...]
    rw = rw_ref[...]
    logits = jax.lax.dot_general(
        rw, xf, (((1,), (1,)), ((), ())),
        preferred_element_type=jnp.float32)
    iota_e = jax.lax.broadcasted_iota(jnp.int32, (_NE, _S), 0)
    v1 = jnp.max(logits, axis=0, keepdims=True)
    i1 = jnp.min(jnp.where(logits == v1, iota_e, _NE), axis=0, keepdims=True)
    m1 = (iota_e == i1)
    l2 = jnp.where(m1, _NEG, logits)
    v2 = jnp.max(l2, axis=0, keepdims=True)
    i2 = jnp.min(jnp.where(l2 == v2, iota_e, _NE), axis=0, keepdims=True)
    m2 = (iota_e == i2)
    a = jnp.exp(v2 - v1)
    w0 = 1.0 / (1.0 + a)

    m1f = m1.astype(jnp.float32)
    m2f = m2.astype(jnp.float32)
    ti = jax.lax.broadcasted_iota(jnp.int32, (_S, _S), 0)
    tj = jax.lax.broadcasted_iota(jnp.int32, (_S, _S), 1)
    lt = (ti < tj).astype(jnp.float32)
    c2 = jnp.concatenate([m1f, m2f], axis=0)
    p12 = jax.lax.dot_general(
        c2, lt, (((1,), (0,)), ((), ())),
        preferred_element_type=jnp.float32)
    p1 = p12[:_NE]
    p2 = p12[_NE:]
    counts1 = jnp.sum(m1f, axis=1, keepdims=True)
    counts2 = jnp.sum(m2f, axis=1, keepdims=True)
    c = counts1 + counts2
    nb = jnp.ceil(c / _BLK)
    ei = jax.lax.broadcasted_iota(jnp.int32, (_NE, _NE), 0)
    ej = jax.lax.broadcasted_iota(jnp.int32, (_NE, _NE), 1)
    lt8 = (ej < ei).astype(jnp.float32)
    snb = jax.lax.dot_general(
        lt8, nb, (((1,), (0,)), ((), ())),
        preferred_element_type=jnp.float32)
    estart = _BLK * snb

    rank0 = jnp.sum(m1f * p1, axis=0, keepdims=True)
    rank1 = jnp.sum(m2f * (p2 + counts1), axis=0, keepdims=True)
    base0 = jnp.sum(m1f * estart, axis=0, keepdims=True)
    base1 = jnp.sum(m2f * estart, axis=0, keepdims=True)
    dest_ref[0:1, :] = (base0 + rank0).astype(jnp.int32)
    dest_ref[1:2, :] = (base1 + rank1).astype(jnp.int32)
    wts_ref[0:1, :] = w0
    wts_ref[1:2, :] = 1.0 - w0

    ends = (snb + nb) * 1.0
    bio = jax.lax.broadcasted_iota(jnp.int32, (_NE, 128), 1).astype(jnp.float32)
    cnt = jnp.sum((bio >= ends).astype(jnp.float32), axis=0, keepdims=True)
    be_ref[...] = jnp.minimum(cnt, _NE - 1.0).astype(jnp.int32)
    total = jnp.sum(nb, axis=0, keepdims=True)
    bv_ref[...] = (bio[0:1, :] < total).astype(jnp.int32)


@jax.jit
def _route(x, router_w):
    return pl.pallas_call(
        _route_body,
        grid=(1,),
        in_specs=[
            pl.BlockSpec((_S, _D), lambda i: (0, 0)),
            pl.BlockSpec((_NE, _D), lambda i: (0, 0)),
        ],
        out_specs=[
            pl.BlockSpec((2, _S), lambda i: (0, 0)),
            pl.BlockSpec((2, _S), lambda i: (0, 0)),
            pl.BlockSpec((1, 128), lambda i: (0, 0)),
            pl.BlockSpec((1, 128), lambda i: (0, 0)),
        ],
        out_shape=[
            jax.ShapeDtypeStruct((2, _S), jnp.int32),
            jax.ShapeDtypeStruct((2, _S), jnp.float32),
            jax.ShapeDtypeStruct((1, 128), jnp.int32),
            jax.ShapeDtypeStruct((1, 128), jnp.int32),
        ],
    )(x, router_w)


_FH = _F // 2


def _expert_half(xs_ref, gw_ref, uw_ref, dw_ref):
    xb = xs_ref[...].astype(jnp.bfloat16)
    g = jax.lax.dot_general(
        xb, gw_ref[0].astype(jnp.bfloat16), (((1,), (1,)), ((), ())),
        preferred_element_type=jnp.float32)
    u = jax.lax.dot_general(
        xb, uw_ref[0].astype(jnp.bfloat16), (((1,), (1,)), ((), ())),
        preferred_element_type=jnp.float32)
    h = (g * jax.nn.sigmoid(g) * u).astype(jnp.bfloat16)
    return jax.lax.dot_general(
        h, dw_ref[0].astype(jnp.bfloat16), (((1,), (1,)), ((), ())),
        preferred_element_type=jnp.float32)


def _ffn1_body(be_ref, bv_ref, xs_ref, gw_ref, uw_ref, dw_ref, part_ref):
    b = pl.program_id(0)

    @pl.when(bv_ref[b] == 1)
    def _():
        part_ref[...] = _expert_half(
            xs_ref, gw_ref, uw_ref, dw_ref).astype(jnp.bfloat16)


def _ffn2_body(be_ref, bv_ref, xs_ref, gw_ref, uw_ref, dw_ref, part_ref,
               yp_ref):
    b = pl.program_id(0)

    @pl.when(bv_ref[b] == 1)
    def _():
        eo = _expert_half(xs_ref, gw_ref, uw_ref, dw_ref)
        yp_ref[...] = part_ref[...].astype(jnp.float32) + eo


def _ffn_specs(f, extra_in=()):
    return pltpu.PrefetchScalarGridSpec(
        num_scalar_prefetch=2,
        grid=(_NB,),
        in_specs=[
            pl.BlockSpec((_BLK, _D), lambda b, be, bv: (b, 0)),
            pl.BlockSpec((1, _FH, _D), lambda b, be, bv, f=f: (be[b], f, 0)),
            pl.BlockSpec((1, _FH, _D), lambda b, be, bv, f=f: (be[b], f, 0)),
            pl.BlockSpec((1, _D, _FH), lambda b, be, bv, f=f: (be[b], 0, f)),
        ] + list(extra_in),
        out_specs=pl.BlockSpec((_BLK, _D), lambda b, be, bv: (b, 0)),
    )


@jax.jit
def _ffn(xs, gw, uw, dw, be, bv):
    part = pl.pallas_call(
        _ffn1_body,
        grid_spec=_ffn_specs(0),
        out_shape=jax.ShapeDtypeStruct((_P, _D), jnp.bfloat16),
        compiler_params=pltpu.CompilerParams(
            dimension_semantics=("arbitrary",),
        ),
    )(be, bv, xs, gw, uw, dw)
    return pl.pallas_call(
        _ffn2_body,
        grid_spec=_ffn_specs(
            1, [pl.BlockSpec((_BLK, _D), lambda b, be, bv: (b, 0))]),
        out_shape=jax.ShapeDtypeStruct((_P, _D), jnp.float32),
        compiler_params=pltpu.CompilerParams(
            dimension_semantics=("arbitrary",),
        ),
    )(be, bv, xs, gw, uw, dw, part)


_MESH = plsc.VectorSubcoreMesh(core_axis_name="c", subcore_axis_name="s")


def _wid():
    return lax.axis_index("s") * 2 + lax.axis_index("c")


def _dispatch_body(x_hbm, dest_hbm, xs_hbm, idx_v, buf0, buf1,
                   rs0, rs1, ws0, ws1):
    w = _wid()
    k = w // 16
    t0 = (w % 16) * 128
    for j in range(4):
        pltpu.sync_copy(dest_hbm.at[k, pl.ds(t0 + j * 32, 32)], idx_v.at[j])
    bufs = (buf0, buf1)
    rsems = (rs0, rs1)
    wsems = (ws0, ws1)
    rd = [None] * 4
    wr = [None] * 4
    rd[0] = pltpu.async_copy(x_hbm.at[pl.ds(t0, 32)], buf0, rs0)
    rd[1] = pltpu.async_copy(x_hbm.at[pl.ds(t0 + 32, 32)], buf1, rs1)
    for j in range(4):
        rd[j].wait()
        wr[j] = pltpu.async_copy(bufs[j % 2], xs_hbm.at[idx_v.at[j]],
                                 wsems[j % 2])
        if j + 2 < 4:
            wr[j].wait()
            rd[j + 2] = pltpu.async_copy(
                x_hbm.at[pl.ds(t0 + (j + 2) * 32, 32)], bufs[j % 2],
                rsems[j % 2])
    wr[2].wait()
    wr[3].wait()


@jax.jit
def _dispatch(x, dest):
    f = functools.partial(
        pl.kernel, mesh=_MESH,
        out_type=jax.ShapeDtypeStruct((_P, _D), jnp.float32),
        scratch_types=[
            pltpu.VMEM((4, 32), jnp.int32),
            pltpu.VMEM((32, _D), jnp.float32),
            pltpu.VMEM((32, _D), jnp.float32),
            pltpu.SemaphoreType.DMA,
            pltpu.SemaphoreType.DMA,
            pltpu.SemaphoreType.DMA,
            pltpu.SemaphoreType.DMA,
        ],
    )(_dispatch_body)
    return f(x, dest)


def _gather2_body(yp_hbm, dest_hbm, r0_hbm, r1_hbm, idx_v, buf0, buf1,
                  rs0, rs1, ws0, ws1):
    w = _wid()
    t0 = w * 64
    for u in range(4):
        k, j = u // 2, u % 2
        pltpu.sync_copy(dest_hbm.at[k, pl.ds(t0 + j * 32, 32)], idx_v.at[u])
    bufs = (buf0, buf1)
    rsems = (rs0, rs1)
    wsems = (ws0, ws1)
    outs = (r0_hbm, r0_hbm, r1_hbm, r1_hbm)
    rd = [None] * 4
    wr = [None] * 4
    rd[0] = pltpu.async_copy(yp_hbm.at[idx_v.at[0]], buf0, rs0)
    rd[1] = pltpu.async_copy(yp_hbm.at[idx_v.at[1]], buf1, rs1)
    for u in range(4):
        j = u % 2
        rd[u].wait()
        wr[u] = pltpu.async_copy(bufs[u % 2],
                                 outs[u].at[pl.ds(t0 + j * 32, 32)],
                                 wsems[u % 2])
        if u + 2 < 4:
            wr[u].wait()
            rd[u + 2] = pltpu.async_copy(yp_hbm.at[idx_v.at[u + 2]],
                                         bufs[u % 2], rsems[u % 2])
    wr[2].wait()
    wr[3].wait()


@jax.jit
def _gather2(yp, dest):
    f = functools.partial(
        pl.kernel, mesh=_MESH,
        out_type=[
            jax.ShapeDtypeStruct((_S, _D), jnp.float32),
            jax.ShapeDtypeStruct((_S, _D), jnp.float32),
        ],
        scratch_types=[
            pltpu.VMEM((4, 32), jnp.int32),
            pltpu.VMEM((32, _D), jnp.float32),
            pltpu.VMEM((32, _D), jnp.float32),
            pltpu.SemaphoreType.DMA,
            pltpu.SemaphoreType.DMA,
            pltpu.SemaphoreType.DMA,
            pltpu.SemaphoreType.DMA,
        ],
    )(_gather2_body)
    return f(yp, dest)


def _mix_body(x_ref, rw_ref, r0_ref, r1_ref, out_ref):
    logits = jax.lax.dot_general(
        x_ref[...], rw_ref[...], (((1,), (1,)), ((), ())),
        preferred_element_type=jnp.float32)
    iota = jax.lax.broadcasted_iota(jnp.int32, logits.shape, 1)
    v1 = jnp.max(logits, axis=1, keepdims=True)
    i1 = jnp.min(jnp.where(logits == v1, iota, _NE), axis=1, keepdims=True)
    l2 = jnp.where(iota == i1, _NEG, logits)
    v2 = jnp.max(l2, axis=1, keepdims=True)
    a = jnp.exp(v2 - v1)
    w0 = 1.0 / (1.0 + a)
    out_ref[...] = w0 * r0_ref[...] + (1.0 - w0) * r1_ref[...]


@jax.jit
def _mix(x, router_w, r0, r1):
    tb = 512
    return pl.pallas_call(
        _mix_body,
        grid=(_S // tb,),
        in_specs=[
            pl.BlockSpec((tb, _D), lambda t: (t, 0)),
            pl.BlockSpec((_NE, _D), lambda t: (0, 0)),
            pl.BlockSpec((tb, _D), lambda t: (t, 0)),
            pl.BlockSpec((tb, _D), lambda t: (t, 0)),
        ],
        out_specs=pl.BlockSpec((tb, _D), lambda t: (t, 0)),
        out_shape=jax.ShapeDtypeStruct((_S, _D), jnp.float32),
    )(x, router_w, r0, r1)


def kernel(hidden_states, router_w, gate_w, up_w, down_w):
    B, S, D = hidden_states.shape
    x = hidden_states.reshape(S, D)
    dest, wts, be, bv = _route(x, router_w)
    be1 = be.reshape(128)
    bv1 = bv.reshape(128)
    xs = _dispatch(x, dest)
    yp = _ffn(xs, gate_w, up_w, down_w, be1, bv1)
    r0, r1 = _gather2(yp, dest)
    out = _mix(x, router_w, r0, r1)
    return out.reshape(B, S, D)
